# initial kernel scaffold (unmeasured)
import jax
import jax.numpy as jnp
from jax import lax
from jax.experimental import pallas as pl
from jax.experimental.pallas import tpu as pltpu

N_DEV = 4
SQ = 2048
SKV_LOCAL = 2048
HQ = 8
DH = 128
DM = HQ * DH
BLK = 64
SCALE = 0.08838834764831843
TQ = 256
PRE_T = 512
NEG = -1e9


def kernel(x, Wq, K_ext, V_ext, Wo):
    def body(x_hbm, wq_ref, k_hbm, v_hbm, wo_ref, out_ref,
             q_scr, kbf, vbf, bias_t, ctx_comm, ml_comm,
             m_run, l_run, acc, send_sems, recv_sems, local_sems):
        my = lax.axis_index("i")
        left = lax.rem(my - 1 + N_DEV, N_DEV)
        right = lax.rem(my + 1, N_DEV)

        barrier = pltpu.get_barrier_semaphore()
        for nbr in (left, right):
            pl.semaphore_signal(barrier, inc=1, device_id=(nbr,),
                                device_id_type=pl.DeviceIdType.MESH)
        pl.semaphore_wait(barrier, 2)

        def pre(x_tile, stage):
            wq_bf = wq_ref[...].astype(jnp.bfloat16)
            for t in range(SQ // PRE_T):
                cp = pltpu.make_async_copy(
                    x_hbm.at[0, pl.ds(t * PRE_T, PRE_T), :], x_tile,
                    local_sems.at[0])
                cp.start()
                cp.wait()
                q = lax.dot_general(
                    x_tile[...].astype(jnp.bfloat16), wq_bf,
                    (((1,), (0,)), ((), ())),
                    preferred_element_type=jnp.float32)
                q_scr[pl.ds(t * PRE_T, PRE_T), :] = q.astype(jnp.bfloat16)
            for t in range(SKV_LOCAL // PRE_T):
                cpk = pltpu.make_async_copy(
                    k_hbm.at[0, pl.ds(t * PRE_T, PRE_T)], stage,
                    local_sems.at[1])
                cpk.start()
                cpk.wait()
                kbf[pl.ds(t * PRE_T, PRE_T), :] = (
                    stage[...].reshape(PRE_T, DM).astype(jnp.bfloat16))
                cpv = pltpu.make_async_copy(
                    v_hbm.at[0, pl.ds(t * PRE_T, PRE_T)], stage,
                    local_sems.at[2])
                cpv.start()
                cpv.wait()
                vbf[pl.ds(t * PRE_T, PRE_T), :] = (
                    stage[...].reshape(PRE_T, DM).astype(jnp.bfloat16))

        pl.run_scoped(pre,
                      x_tile=pltpu.VMEM((PRE_T, DM), jnp.float32),
                      stage=pltpu.VMEM((PRE_T, HQ, DH), jnp.float32))

        kb_off = my * (SKV_LOCAL // BLK)
        for t in range(SQ // TQ):
            r0 = t * TQ
            qb = (lax.broadcasted_iota(jnp.int32, (TQ, SKV_LOCAL), 0)
                  // BLK) + (r0 // BLK)
            kb = (lax.broadcasted_iota(jnp.int32, (TQ, SKV_LOCAL), 1)
                  // BLK) + kb_off
            mask = (qb == kb) | (kb == 0) | (lax.rem(qb + kb, 3) == 0)
            bias_t[...] = jnp.where(mask, 0.0, NEG)

            def head_body(h, carry):
                qth = q_scr[pl.ds(r0, TQ), pl.ds(h * DH, DH)]
                kh = kbf[:, pl.ds(h * DH, DH)]
                vh = vbf[:, pl.ds(h * DH, DH)]
                s = lax.dot_general(qth, kh, (((1,), (1,)), ((), ())),
                                    preferred_element_type=jnp.float32)
                s = s * SCALE + bias_t[...]
                m = jnp.max(s, axis=1, keepdims=True)
                p = jnp.exp(s - m)
                l = jnp.sum(p, axis=1, keepdims=True)
                ctx = lax.dot_general(
                    p.astype(jnp.bfloat16), vh, (((1,), (0,)), ((), ())),
                    preferred_element_type=jnp.float32)
                acc[pl.ds(r0, TQ), pl.ds(h, 1), :] = ctx[:, None, :]
                ctx_comm[0, pl.ds(r0, TQ), pl.ds(h, 1), :] = (
                    ctx[:, None, :].astype(jnp.bfloat16))
                ml_comm[0, pl.ds(r0, TQ), pl.ds(h, 1)] = m
                ml_comm[0, pl.ds(r0, TQ), pl.ds(h + HQ, 1)] = l
                m_run[pl.ds(r0, TQ), pl.ds(h, 1)] = m
                l_run[pl.ds(r0, TQ), pl.ds(h, 1)] = l
                return carry

            lax.fori_loop(0, HQ, head_body, 0)

        for hop in range(N_DEV - 1):
            s_slot = hop % 2
            r_slot = (hop + 1) % 2
            rd_ctx = pltpu.make_async_remote_copy(
                src_ref=ctx_comm.at[s_slot], dst_ref=ctx_comm.at[r_slot],
                send_sem=send_sems.at[0, s_slot],
                recv_sem=recv_sems.at[0, r_slot],
                device_id=(right,), device_id_type=pl.DeviceIdType.MESH)
            rd_ml = pltpu.make_async_remote_copy(
                src_ref=ml_comm.at[s_slot], dst_ref=ml_comm.at[r_slot],
                send_sem=send_sems.at[1, s_slot],
                recv_sem=recv_sems.at[1, r_slot],
                device_id=(right,), device_id_type=pl.DeviceIdType.MESH)
            rd_ctx.start()
            rd_ml.start()
            rd_ctx.wait()
            rd_ml.wait()
            for c in range(2):
                rows = pl.ds(c * (SQ // 2), SQ // 2)
                m_in = ml_comm[r_slot, rows, 0:HQ]
                l_in = ml_comm[r_slot, rows, HQ:2 * HQ]
                m_old = m_run[rows, :]
                m_new = jnp.maximum(m_old, m_in)
                a_old = jnp.exp(m_old - m_new)
                a_in = jnp.exp(m_in - m_new)
                l_run[rows, :] = l_run[rows, :] * a_old + l_in * a_in
                m_run[rows, :] = m_new
                acc[rows, :, :] = (
                    acc[rows, :, :] * a_old[:, :, None]
                    + ctx_comm[r_slot, rows, :, :].astype(jnp.float32)
                    * a_in[:, :, None])

        wo_bf = wo_ref[...].astype(jnp.bfloat16)
        for c in range(SQ // PRE_T):
            rows = pl.ds(c * PRE_T, PRE_T)
            z = acc[rows, :, :] / l_run[rows, :][:, :, None]
            z2 = z.reshape(PRE_T, DM).astype(jnp.bfloat16)
            o = lax.dot_general(z2, wo_bf, (((1,), (0,)), ((), ())),
                                preferred_element_type=jnp.float32)
            out_ref[0, rows, :] = o

    return pl.pallas_call(
        body,
        out_shape=jax.ShapeDtypeStruct((1, SQ, DM), jnp.float32),
        in_specs=[
            pl.BlockSpec(memory_space=pltpu.ANY),
            pl.BlockSpec(memory_space=pltpu.VMEM),
            pl.BlockSpec(memory_space=pltpu.ANY),
            pl.BlockSpec(memory_space=pltpu.ANY),
            pl.BlockSpec(memory_space=pltpu.VMEM),
        ],
        out_specs=pl.BlockSpec(memory_space=pltpu.VMEM),
        scratch_shapes=[
            pltpu.VMEM((SQ, DM), jnp.bfloat16),
            pltpu.VMEM((SKV_LOCAL, DM), jnp.bfloat16),
            pltpu.VMEM((SKV_LOCAL, DM), jnp.bfloat16),
            pltpu.VMEM((TQ, SKV_LOCAL), jnp.float32),
            pltpu.VMEM((2, SQ, HQ, DH), jnp.bfloat16),
            pltpu.VMEM((2, SQ, 2 * HQ), jnp.float32),
            pltpu.VMEM((SQ, HQ), jnp.float32),
            pltpu.VMEM((SQ, HQ), jnp.float32),
            pltpu.VMEM((SQ, HQ, DH), jnp.float32),
            pltpu.SemaphoreType.DMA((2, 2)),
            pltpu.SemaphoreType.DMA((2, 2)),
            pltpu.SemaphoreType.DMA((3,)),
        ],
        compiler_params=pltpu.CompilerParams(collective_id=0),
    )(x, Wq, K_ext, V_ext, Wo)


# baseline (device time: 338932 ns/iter reference)
import jax
import jax.numpy as jnp
from jax import lax
from jax.experimental import pallas as pl
from jax.experimental.pallas import tpu as pltpu

N_DEV = 4
SQ = 2048
SKV_LOCAL = 2048
HQ = 8
DH = 128
DM = HQ * DH
BLK = 64
SCALE = 0.08838834764831843
TQ = 256
PRE_T = 512
NEG = -1e9


def kernel(x, Wq, K_ext, V_ext, Wo):
    def body(x_hbm, wq_ref, k_hbm, v_hbm, wo_ref, out_ref,
             q_scr, kbf, vbf, bias_t, ctx_comm, ml_comm,
             m_run, l_run, acc, send_sems, recv_sems, local_sems):
        my = lax.axis_index("i")
        left = lax.rem(my - 1 + N_DEV, N_DEV)
        right = lax.rem(my + 1, N_DEV)

        barrier = pltpu.get_barrier_semaphore()
        for nbr in (left, right):
            pl.semaphore_signal(barrier, inc=1, device_id=(nbr,),
                                device_id_type=pl.DeviceIdType.MESH)
        pl.semaphore_wait(barrier, 2)

        def pre(x_tile, stage):
            def q_step(t, carry):
                cp = pltpu.make_async_copy(
                    x_hbm.at[0, pl.ds(t * PRE_T, PRE_T), :], x_tile,
                    local_sems.at[0])
                cp.start()
                cp.wait()
                wq_bf = wq_ref[...].astype(jnp.bfloat16)
                q = lax.dot_general(
                    x_tile[...].astype(jnp.bfloat16), wq_bf,
                    (((1,), (0,)), ((), ())),
                    preferred_element_type=jnp.float32)
                q_scr[pl.ds(t * PRE_T, PRE_T), :] = q.astype(jnp.bfloat16)
                return carry

            lax.fori_loop(0, SQ // PRE_T, q_step, 0)

            def kv_step(t, carry):
                cpk = pltpu.make_async_copy(
                    k_hbm.at[0, pl.ds(t * PRE_T, PRE_T)], stage,
                    local_sems.at[1])
                cpk.start()
                cpk.wait()
                kbf[pl.ds(t * PRE_T, PRE_T), :] = (
                    stage[...].reshape(PRE_T, DM).astype(jnp.bfloat16))
                cpv = pltpu.make_async_copy(
                    v_hbm.at[0, pl.ds(t * PRE_T, PRE_T)], stage,
                    local_sems.at[2])
                cpv.start()
                cpv.wait()
                vbf[pl.ds(t * PRE_T, PRE_T), :] = (
                    stage[...].reshape(PRE_T, DM).astype(jnp.bfloat16))
                return carry

            lax.fori_loop(0, SKV_LOCAL // PRE_T, kv_step, 0)

        pl.run_scoped(pre,
                      x_tile=pltpu.VMEM((PRE_T, DM), jnp.float32),
                      stage=pltpu.VMEM((PRE_T, HQ, DH), jnp.float32))

        kb_off = my * (SKV_LOCAL // BLK)
        lane_h = lax.broadcasted_iota(jnp.int32, (TQ, HQ), 1)

        def tile_step(t, carry):
            r0 = t * TQ
            qb = (lax.broadcasted_iota(jnp.int32, (TQ, SKV_LOCAL), 0)
                  // BLK) + (r0 // BLK)
            kb = (lax.broadcasted_iota(jnp.int32, (TQ, SKV_LOCAL), 1)
                  // BLK) + kb_off
            mask = (qb == kb) | (kb == 0) | (lax.rem(qb + kb, 3) == 0)
            bias_t[...] = jnp.where(mask, 0.0, NEG).astype(jnp.bfloat16)
            rows = pl.ds(r0, TQ)

            def head_step(h, ml):
                m_tile, l_tile = ml
                cols = pl.ds(h * DH, DH)
                qth = q_scr[rows, cols]
                kh = kbf[:, cols]
                vh = vbf[:, cols]
                s = lax.dot_general(qth, kh, (((1,), (1,)), ((), ())),
                                    preferred_element_type=jnp.float32)
                s = s * SCALE + bias_t[...].astype(jnp.float32)
                m = jnp.max(s, axis=1, keepdims=True)
                e = jnp.exp(s - m)
                l = jnp.sum(e, axis=1, keepdims=True)
                ctx = lax.dot_general(
                    e.astype(jnp.bfloat16), vh, (((1,), (0,)), ((), ())),
                    preferred_element_type=jnp.float32)
                acc[rows, cols] = ctx
                ctx_comm[0, rows, cols] = ctx.astype(jnp.bfloat16)
                m_tile = jnp.where(lane_h == h, m, m_tile)
                l_tile = jnp.where(lane_h == h, l, l_tile)
                return (m_tile, l_tile)

            zeros = jnp.zeros((TQ, HQ), jnp.float32)
            m_tile, l_tile = lax.fori_loop(0, HQ, head_step, (zeros, zeros))
            m_run[rows, :] = m_tile
            l_run[rows, :] = l_tile
            ml_comm[0, rows, 0:HQ] = m_tile
            ml_comm[0, rows, HQ:2 * HQ] = l_tile
            return carry

        lax.fori_loop(0, SQ // TQ, tile_step, 0)

        for hop in range(N_DEV - 1):
            s_slot = hop % 2
            r_slot = (hop + 1) % 2
            rd_ctx = pltpu.make_async_remote_copy(
                src_ref=ctx_comm.at[s_slot], dst_ref=ctx_comm.at[r_slot],
                send_sem=send_sems.at[0, s_slot],
                recv_sem=recv_sems.at[0, r_slot],
                device_id=(right,), device_id_type=pl.DeviceIdType.MESH)
            rd_ml = pltpu.make_async_remote_copy(
                src_ref=ml_comm.at[s_slot], dst_ref=ml_comm.at[r_slot],
                send_sem=send_sems.at[1, s_slot],
                recv_sem=recv_sems.at[1, r_slot],
                device_id=(right,), device_id_type=pl.DeviceIdType.MESH)
            rd_ctx.start()
            rd_ml.start()
            rd_ctx.wait()
            rd_ml.wait()

            def merge_step(c, carry, r_slot=r_slot):
                rows = pl.ds(c * TQ, TQ)
                m_in = ml_comm[r_slot, rows, 0:HQ]
                l_in = ml_comm[r_slot, rows, HQ:2 * HQ]
                m_old = m_run[rows, :]
                m_new = jnp.maximum(m_old, m_in)
                a_old = jnp.exp(m_old - m_new)
                a_in = jnp.exp(m_in - m_new)
                l_run[rows, :] = l_run[rows, :] * a_old + l_in * a_in
                m_run[rows, :] = m_new
                for h in range(HQ):
                    cols = pl.ds(h * DH, DH)
                    acc[rows, cols] = (
                        acc[rows, cols] * a_old[:, h:h + 1]
                        + ctx_comm[r_slot, rows, cols].astype(jnp.float32)
                        * a_in[:, h:h + 1])
                return carry

            lax.fori_loop(0, SQ // TQ, merge_step, 0)

        def out_phase(z_stage, o_stage):
            def out_step(c, carry):
                rows = pl.ds(c * TQ, TQ)
                inv_l = 1.0 / l_run[rows, :]
                for h in range(HQ):
                    cols = pl.ds(h * DH, DH)
                    z_stage[:, cols] = (
                        acc[rows, cols] * inv_l[:, h:h + 1]
                    ).astype(jnp.bfloat16)
                wo_bf = wo_ref[...].astype(jnp.bfloat16)
                o = lax.dot_general(z_stage[...], wo_bf,
                                    (((1,), (0,)), ((), ())),
                                    preferred_element_type=jnp.float32)
                o_stage[...] = o
                cp = pltpu.make_async_copy(
                    o_stage, out_ref.at[0, rows, :], local_sems.at[0])
                cp.start()
                cp.wait()
                return carry

            lax.fori_loop(0, SQ // TQ, out_step, 0)

        pl.run_scoped(out_phase,
                      z_stage=pltpu.VMEM((TQ, DM), jnp.bfloat16),
                      o_stage=pltpu.VMEM((TQ, DM), jnp.float32))

    return pl.pallas_call(
        body,
        out_shape=jax.ShapeDtypeStruct((1, SQ, DM), jnp.float32),
        in_specs=[
            pl.BlockSpec(memory_space=pltpu.MemorySpace.HBM),
            pl.BlockSpec(memory_space=pltpu.MemorySpace.VMEM),
            pl.BlockSpec(memory_space=pltpu.MemorySpace.HBM),
            pl.BlockSpec(memory_space=pltpu.MemorySpace.HBM),
            pl.BlockSpec(memory_space=pltpu.MemorySpace.VMEM),
        ],
        out_specs=pl.BlockSpec(memory_space=pltpu.MemorySpace.HBM),
        scratch_shapes=[
            pltpu.VMEM((SQ, DM), jnp.bfloat16),
            pltpu.VMEM((SKV_LOCAL, DM), jnp.bfloat16),
            pltpu.VMEM((SKV_LOCAL, DM), jnp.bfloat16),
            pltpu.VMEM((TQ, SKV_LOCAL), jnp.bfloat16),
            pltpu.VMEM((2, SQ, DM), jnp.bfloat16),
            pltpu.VMEM((2, SQ, 2 * HQ), jnp.float32),
            pltpu.VMEM((SQ, HQ), jnp.float32),
            pltpu.VMEM((SQ, HQ), jnp.float32),
            pltpu.VMEM((SQ, DM), jnp.float32),
            pltpu.SemaphoreType.DMA((2, 2)),
            pltpu.SemaphoreType.DMA((2, 2)),
            pltpu.SemaphoreType.DMA((3,)),
        ],
        compiler_params=pltpu.CompilerParams(
            collective_id=0, vmem_limit_bytes=40 * 1024 * 1024),
    )(x, Wq, K_ext, V_ext, Wo)
